# baseline (speedup 1.0000x reference)
import functools

import jax
import jax.numpy as jnp
import numpy as np
from jax.experimental import pallas as pl
from jax.experimental.pallas import tpu as pltpu

N_E = 1024
E_DIM = 256
N_CHANNEL = 4
D_MODEL = 1024
BETA = 0.25

_TOK_BLK = 128
_ROW_BLK = _TOK_BLK * N_CHANNEL

_TINY = np.float32(1.1754944e-38)
_KEY0 = 0
_KEY1 = 42


def _rotl(x, r):
    return jax.lax.shift_left(x, jnp.uint32(r)) | jax.lax.shift_right_logical(
        x, jnp.uint32(32 - r))


def _threefry2x32(x1):
    rot1 = (13, 15, 26, 6)
    rot2 = (17, 29, 16, 24)
    ks0 = jnp.uint32(_KEY0)
    ks1 = jnp.uint32(_KEY1)
    ks2 = ks0 ^ ks1 ^ jnp.uint32(0x1BD11BDA)
    ks = (ks0, ks1, ks2)
    x0 = jnp.zeros_like(x1) + ks0
    for i in range(5):
        for r in (rot1 if i % 2 == 0 else rot2):
            x0 = x0 + x1
            x1 = _rotl(x1, r)
            x1 = x0 ^ x1
        x0 = x0 + ks[(i + 1) % 3]
        x1 = x1 + (ks[(i + 2) % 3] + jnp.uint32(i + 1))
    return x0, x1


def _gumbel_from_counts(x1):
    b0, b1 = _threefry2x32(x1)
    fbits = jax.lax.shift_right_logical(b0 ^ b1, jnp.uint32(9)) | jnp.uint32(
        0x3F800000)
    f = jax.lax.bitcast_convert_type(fbits, jnp.float32) - jnp.float32(1.0)
    u = jnp.maximum(f, _TINY)
    return -jnp.log(-jnp.log(u))


def _vq_kernel(pk_ref, z_ref, fc0_w_ref, fc0_b_ref, fc1_w_ref, fc1_b_ref,
               emb_ref, emb_m2_ref, zq_ref, loss_ref):
    i = pl.program_id(0)

    emb = emb_ref[...]
    e_blk = jax.lax.dot_general(
        z_ref[...], fc0_w_ref[...], (((1,), (1,)), ((), ())),
        preferred_element_type=jnp.float32) + fc0_b_ref[...]
    cz = e_blk.reshape(_ROW_BLK, E_DIM)

    s_z = jnp.sum(cz * cz, axis=1, keepdims=True)
    s_e = jnp.sum(emb * emb, axis=1)[None, :]
    cross_m2 = jax.lax.dot_general(
        cz, emb_m2_ref[...], (((1,), (1,)), ((), ())),
        preferred_element_type=jnp.float32)
    logits = (s_z + s_e) + cross_m2
    ls = logits - jnp.max(logits, axis=1, keepdims=True)

    g = _gumbel_from_counts(pk_ref[...] + jnp.uint32(i * (_ROW_BLK * N_E)))
    y = g + ls
    m = jnp.max(y, axis=1, keepdims=True)
    colf = jax.lax.broadcasted_iota(jnp.int32, (_ROW_BLK, N_E), 1)
    idx = jnp.min(jnp.where(y == m, colf, jnp.int32(N_E)), axis=1)
    sel = colf == idx[:, None]

    part = jnp.sum(jnp.where(sel, logits, jnp.float32(0.0)))

    @pl.when(i == 0)
    def _():
        loss_ref[...] = jnp.zeros((1, 1), jnp.float32)

    loss_ref[...] += part.reshape(1, 1)

    czq = jax.lax.dot_general(
        sel.astype(jnp.float32), emb, (((1,), (0,)), ((), ())),
        preferred_element_type=jnp.float32)

    q_blk = czq.reshape(_TOK_BLK, N_CHANNEL * E_DIM)
    zq_ref[...] = jax.lax.dot_general(
        q_blk, fc1_w_ref[...], (((1,), (1,)), ((), ())),
        preferred_element_type=jnp.float32) + fc1_b_ref[...]


@functools.partial(jax.jit, static_argnums=())
def kernel(z, fc0_w, fc0_b, fc1_w, fc1_b, emb):
    n_batch, n_seq, d_model = z.shape
    n_tok = n_batch * n_seq
    z2 = z.reshape(n_tok, d_model)
    grid = (n_tok // _TOK_BLK,)

    pk = (jnp.arange(_ROW_BLK * N_E, dtype=jnp.uint32) +
          jnp.uint32(_KEY1)).reshape(_ROW_BLK, N_E)

    zq, loss_sum = pl.pallas_call(
        _vq_kernel,
        grid=grid,
        in_specs=[
            pl.BlockSpec((_ROW_BLK, N_E), lambda i: (0, 0)),
            pl.BlockSpec((_TOK_BLK, d_model), lambda i: (i, 0)),
            pl.BlockSpec((D_MODEL, D_MODEL), lambda i: (0, 0)),
            pl.BlockSpec((1, D_MODEL), lambda i: (0, 0)),
            pl.BlockSpec((D_MODEL, D_MODEL), lambda i: (0, 0)),
            pl.BlockSpec((1, D_MODEL), lambda i: (0, 0)),
            pl.BlockSpec((N_E, E_DIM), lambda i: (0, 0)),
            pl.BlockSpec((N_E, E_DIM), lambda i: (0, 0)),
        ],
        out_specs=[
            pl.BlockSpec((_TOK_BLK, d_model), lambda i: (i, 0)),
            pl.BlockSpec((1, 1), lambda i: (0, 0)),
        ],
        out_shape=[
            jax.ShapeDtypeStruct((n_tok, d_model), jnp.float32),
            jax.ShapeDtypeStruct((1, 1), jnp.float32),
        ],
        compiler_params=pltpu.CompilerParams(
            dimension_semantics=("arbitrary",),
        ),
    )(pk, z2, fc0_w, fc0_b.reshape(1, -1), fc1_w, fc1_b.reshape(1, -1), emb,
      jnp.float32(-2.0) * emb)

    mean = loss_sum[0, 0] / jnp.float32(n_tok * N_CHANNEL * E_DIM)
    loss = mean + jnp.float32(BETA) * mean
    return (loss, zq.reshape(n_batch, n_seq, d_model))


# speedup vs baseline: 3.1704x; 3.1704x over previous
import functools

import jax
import jax.numpy as jnp
import numpy as np
from jax.experimental import pallas as pl
from jax.experimental.pallas import tpu as pltpu

N_E = 1024
E_DIM = 256
N_CHANNEL = 4
D_MODEL = 1024
BETA = 0.25

_TOK_BLK = 256
_ROW_BLK = _TOK_BLK * N_CHANNEL

_TINY = np.float32(1.1754944e-38)


@functools.lru_cache(maxsize=1)
def _threefry_bits_table(n_rows: int, n_cols: int):
    rot1 = (13, 15, 26, 6)
    rot2 = (17, 29, 16, 24)
    ks = (np.uint32(0), np.uint32(42),
          np.uint32(0) ^ np.uint32(42) ^ np.uint32(0x1BD11BDA))
    with np.errstate(over="ignore"):
        x1 = np.arange(n_rows * n_cols, dtype=np.uint32) + ks[1]
        x0 = np.full_like(x1, ks[0])
        for i in range(5):
            for r in (rot1 if i % 2 == 0 else rot2):
                x0 = x0 + x1
                x1 = (x1 << np.uint32(r)) | (x1 >> np.uint32(32 - r))
                x1 ^= x0
            x0 = x0 + ks[(i + 1) % 3]
            x1 = x1 + (ks[(i + 2) % 3] + np.uint32(i + 1))
    return (x0 ^ x1).reshape(n_rows, n_cols)


def _vq_kernel(bits_ref, z_ref, fc0_w_ref, fc0_b_ref, fc1_w_ref, fc1_b_ref,
               emb_ref, emb_m2_ref, zq_ref, loss_ref):
    i = pl.program_id(0)

    emb = emb_ref[...]
    e_blk = jax.lax.dot_general(
        z_ref[...], fc0_w_ref[...], (((1,), (1,)), ((), ())),
        preferred_element_type=jnp.float32) + fc0_b_ref[...]
    cz = e_blk.reshape(_ROW_BLK, E_DIM)

    s_z = jnp.sum(cz * cz, axis=1, keepdims=True)
    s_e = jnp.sum(emb * emb, axis=1)[None, :]
    cross_m2 = jax.lax.dot_general(
        cz, emb_m2_ref[...], (((1,), (1,)), ((), ())),
        preferred_element_type=jnp.float32)
    logits = (s_z + s_e) + cross_m2
    ls = logits - jnp.max(logits, axis=1, keepdims=True)

    fbits = jax.lax.shift_right_logical(
        bits_ref[...], jnp.uint32(9)) | jnp.uint32(0x3F800000)
    f = jax.lax.bitcast_convert_type(fbits, jnp.float32) - jnp.float32(1.0)
    g = -jnp.log(-jnp.log(jnp.maximum(f, _TINY)))

    y = g + ls
    m = jnp.max(y, axis=1, keepdims=True)
    colf = jax.lax.broadcasted_iota(jnp.int32, (_ROW_BLK, N_E), 1)
    idx = jnp.min(jnp.where(y == m, colf, jnp.int32(N_E)), axis=1)
    sel = colf == idx[:, None]

    part = jnp.sum(jnp.where(sel, logits, jnp.float32(0.0)))

    @pl.when(i == 0)
    def _():
        loss_ref[...] = jnp.zeros((1, 1), jnp.float32)

    loss_ref[...] += part.reshape(1, 1)

    czq = jax.lax.dot_general(
        sel.astype(jnp.float32), emb, (((1,), (0,)), ((), ())),
        preferred_element_type=jnp.float32)

    q_blk = czq.reshape(_TOK_BLK, N_CHANNEL * E_DIM)
    zq_ref[...] = jax.lax.dot_general(
        q_blk, fc1_w_ref[...], (((1,), (1,)), ((), ())),
        preferred_element_type=jnp.float32) + fc1_b_ref[...]


@functools.partial(jax.jit, static_argnums=())
def kernel(z, fc0_w, fc0_b, fc1_w, fc1_b, emb):
    n_batch, n_seq, d_model = z.shape
    n_tok = n_batch * n_seq
    n_rows = n_tok * N_CHANNEL
    z2 = z.reshape(n_tok, d_model)
    grid = (n_tok // _TOK_BLK,)

    bits = jnp.asarray(_threefry_bits_table(n_rows, N_E))

    zq, loss_sum = pl.pallas_call(
        _vq_kernel,
        grid=grid,
        in_specs=[
            pl.BlockSpec((_ROW_BLK, N_E), lambda i: (i, 0)),
            pl.BlockSpec((_TOK_BLK, d_model), lambda i: (i, 0)),
            pl.BlockSpec((D_MODEL, D_MODEL), lambda i: (0, 0)),
            pl.BlockSpec((1, D_MODEL), lambda i: (0, 0)),
            pl.BlockSpec((D_MODEL, D_MODEL), lambda i: (0, 0)),
            pl.BlockSpec((1, D_MODEL), lambda i: (0, 0)),
            pl.BlockSpec((N_E, E_DIM), lambda i: (0, 0)),
            pl.BlockSpec((N_E, E_DIM), lambda i: (0, 0)),
        ],
        out_specs=[
            pl.BlockSpec((_TOK_BLK, d_model), lambda i: (i, 0)),
            pl.BlockSpec((1, 1), lambda i: (0, 0)),
        ],
        out_shape=[
            jax.ShapeDtypeStruct((n_tok, d_model), jnp.float32),
            jax.ShapeDtypeStruct((1, 1), jnp.float32),
        ],
        compiler_params=pltpu.CompilerParams(
            dimension_semantics=("arbitrary",),
        ),
    )(bits, z2, fc0_w, fc0_b.reshape(1, -1), fc1_w, fc1_b.reshape(1, -1),
      emb, jnp.float32(-2.0) * emb)

    mean = loss_sum[0, 0] / jnp.float32(n_tok * N_CHANNEL * E_DIM)
    loss = mean + jnp.float32(BETA) * mean
    return (loss, zq.reshape(n_batch, n_seq, d_model))


# speedup vs baseline: 3.6048x; 1.1370x over previous
import functools

import jax
import jax.numpy as jnp
import numpy as np
from jax.experimental import pallas as pl
from jax.experimental.pallas import tpu as pltpu

N_E = 1024
E_DIM = 256
N_CHANNEL = 4
D_MODEL = 1024
BETA = 0.25

_TOK_BLK = 256
_ROW_BLK = _TOK_BLK * N_CHANNEL

_TINY = np.float32(1.1754944e-38)


@functools.lru_cache(maxsize=1)
def _threefry_bits_table(n_rows: int, n_cols: int):
    rot1 = (13, 15, 26, 6)
    rot2 = (17, 29, 16, 24)
    ks = (np.uint32(0), np.uint32(42),
          np.uint32(0) ^ np.uint32(42) ^ np.uint32(0x1BD11BDA))
    with np.errstate(over="ignore"):
        x1 = np.arange(n_rows * n_cols, dtype=np.uint32) + ks[1]
        x0 = np.full_like(x1, ks[0])
        for i in range(5):
            for r in (rot1 if i % 2 == 0 else rot2):
                x0 = x0 + x1
                x1 = (x1 << np.uint32(r)) | (x1 >> np.uint32(32 - r))
                x1 ^= x0
            x0 = x0 + ks[(i + 1) % 3]
            x1 = x1 + (ks[(i + 2) % 3] + np.uint32(i + 1))
    bits = x0 ^ x1
    f = ((bits >> np.uint32(9)) | np.uint32(0x3F800000)).view(np.float32)
    return (f - np.float32(1.0)).reshape(n_rows, n_cols)


def _vq_kernel(bits_ref, z_ref, fc0_w_ref, fc0_b_ref, fc1_w_ref, fc1_b_ref,
               emb_ref, emb_m2_ref, zq_ref, loss_ref):
    i = pl.program_id(0)

    emb = emb_ref[...]
    e_blk = jax.lax.dot_general(
        z_ref[...], fc0_w_ref[...], (((1,), (1,)), ((), ())),
        preferred_element_type=jnp.float32) + fc0_b_ref[...]
    cz = e_blk.reshape(_ROW_BLK, E_DIM)

    s_z = jnp.sum(cz * cz, axis=1, keepdims=True)
    s_e = jnp.sum(emb * emb, axis=1)[None, :]
    cross_m2 = jax.lax.dot_general(
        cz, emb_m2_ref[...], (((1,), (1,)), ((), ())),
        preferred_element_type=jnp.float32)
    logits = (s_z + s_e) + cross_m2
    ls = logits - jnp.max(logits, axis=1, keepdims=True)

    g = -jnp.log(-jnp.log(jnp.maximum(bits_ref[...], _TINY)))

    y = g + ls
    m = jnp.max(y, axis=1, keepdims=True)
    colf = jax.lax.broadcasted_iota(jnp.int32, (_ROW_BLK, N_E), 1)
    idx = jnp.min(jnp.where(y == m, colf, jnp.int32(N_E)), axis=1)
    sel = colf == idx[:, None]

    czq = jax.lax.dot_general(
        sel.astype(jnp.float32), emb, (((1,), (0,)), ((), ())),
        preferred_element_type=jnp.float32)

    d = czq - cz
    part = jnp.sum(d * d)

    @pl.when(i == 0)
    def _():
        loss_ref[...] = jnp.zeros((1, 1), jnp.float32)

    loss_ref[...] += part.reshape(1, 1)

    q_blk = czq.reshape(_TOK_BLK, N_CHANNEL * E_DIM)
    zq_ref[...] = jax.lax.dot_general(
        q_blk, fc1_w_ref[...], (((1,), (1,)), ((), ())),
        preferred_element_type=jnp.float32) + fc1_b_ref[...]


@functools.partial(jax.jit, static_argnums=())
def kernel(z, fc0_w, fc0_b, fc1_w, fc1_b, emb):
    n_batch, n_seq, d_model = z.shape
    n_tok = n_batch * n_seq
    n_rows = n_tok * N_CHANNEL
    z2 = z.reshape(n_tok, d_model)
    grid = (n_tok // _TOK_BLK,)

    bits = jnp.asarray(_threefry_bits_table(n_rows, N_E))

    zq, loss_sum = pl.pallas_call(
        _vq_kernel,
        grid=grid,
        in_specs=[
            pl.BlockSpec((_ROW_BLK, N_E), lambda i: (i, 0)),
            pl.BlockSpec((_TOK_BLK, d_model), lambda i: (i, 0)),
            pl.BlockSpec((D_MODEL, D_MODEL), lambda i: (0, 0)),
            pl.BlockSpec((1, D_MODEL), lambda i: (0, 0)),
            pl.BlockSpec((D_MODEL, D_MODEL), lambda i: (0, 0)),
            pl.BlockSpec((1, D_MODEL), lambda i: (0, 0)),
            pl.BlockSpec((N_E, E_DIM), lambda i: (0, 0)),
            pl.BlockSpec((N_E, E_DIM), lambda i: (0, 0)),
        ],
        out_specs=[
            pl.BlockSpec((_TOK_BLK, d_model), lambda i: (i, 0)),
            pl.BlockSpec((1, 1), lambda i: (0, 0)),
        ],
        out_shape=[
            jax.ShapeDtypeStruct((n_tok, d_model), jnp.float32),
            jax.ShapeDtypeStruct((1, 1), jnp.float32),
        ],
        compiler_params=pltpu.CompilerParams(
            dimension_semantics=("arbitrary",),
        ),
    )(bits, z2, fc0_w, fc0_b.reshape(1, -1), fc1_w, fc1_b.reshape(1, -1),
      emb, jnp.float32(-2.0) * emb)

    mean = loss_sum[0, 0] / jnp.float32(n_tok * N_CHANNEL * E_DIM)
    loss = mean + jnp.float32(BETA) * mean
    return (loss, zq.reshape(n_batch, n_seq, d_model))


# speedup vs baseline: 3.8194x; 1.0595x over previous
import functools

import jax
import jax.numpy as jnp
import numpy as np
from jax.experimental import pallas as pl
from jax.experimental.pallas import tpu as pltpu

N_E = 1024
E_DIM = 256
N_CHANNEL = 4
D_MODEL = 1024
BETA = 0.25

_TOK_BLK = 512
_ROW_BLK = _TOK_BLK * N_CHANNEL

_TINY = np.float32(1.1754944e-38)


@functools.lru_cache(maxsize=1)
def _threefry_bits_table(n_rows: int, n_cols: int):
    rot1 = (13, 15, 26, 6)
    rot2 = (17, 29, 16, 24)
    ks = (np.uint32(0), np.uint32(42),
          np.uint32(0) ^ np.uint32(42) ^ np.uint32(0x1BD11BDA))
    with np.errstate(over="ignore"):
        x1 = np.arange(n_rows * n_cols, dtype=np.uint32) + ks[1]
        x0 = np.full_like(x1, ks[0])
        for i in range(5):
            for r in (rot1 if i % 2 == 0 else rot2):
                x0 = x0 + x1
                x1 = (x1 << np.uint32(r)) | (x1 >> np.uint32(32 - r))
                x1 ^= x0
            x0 = x0 + ks[(i + 1) % 3]
            x1 = x1 + (ks[(i + 2) % 3] + np.uint32(i + 1))
    bits = x0 ^ x1
    f = ((bits >> np.uint32(9)) | np.uint32(0x3F800000)).view(np.float32)
    return (f - np.float32(1.0)).reshape(n_rows, n_cols)


def _vq_kernel(bits_ref, z_ref, fc0_w_ref, fc0_b_ref, fc1_w_ref, fc1_b_ref,
               emb_ref, emb_m2_ref, zq_ref, loss_ref):
    i = pl.program_id(0)

    emb = emb_ref[...]
    e_blk = jax.lax.dot_general(
        z_ref[...], fc0_w_ref[...], (((1,), (1,)), ((), ())),
        preferred_element_type=jnp.float32) + fc0_b_ref[...]
    cz = e_blk.reshape(_ROW_BLK, E_DIM)

    s_z = jnp.sum(cz * cz, axis=1, keepdims=True)
    s_e = jnp.sum(emb * emb, axis=1)[None, :]
    cross_m2 = jax.lax.dot_general(
        cz, emb_m2_ref[...], (((1,), (1,)), ((), ())),
        preferred_element_type=jnp.float32)
    logits = (s_z + s_e) + cross_m2
    ls = logits - jnp.max(logits, axis=1, keepdims=True)

    g = -jnp.log(-jnp.log(jnp.maximum(bits_ref[...], _TINY)))

    y = g + ls
    m = jnp.max(y, axis=1, keepdims=True)
    colf = jax.lax.broadcasted_iota(jnp.int32, (_ROW_BLK, N_E), 1)
    idx = jnp.min(jnp.where(y == m, colf, jnp.int32(N_E)), axis=1)
    sel = colf == idx[:, None]

    czq = jax.lax.dot_general(
        sel.astype(jnp.float32), emb, (((1,), (0,)), ((), ())),
        preferred_element_type=jnp.float32)

    d = czq - cz
    part = jnp.sum(d * d)

    @pl.when(i == 0)
    def _():
        loss_ref[...] = jnp.zeros((1, 1), jnp.float32)

    loss_ref[...] += part.reshape(1, 1)

    q_blk = czq.reshape(_TOK_BLK, N_CHANNEL * E_DIM)
    zq_ref[...] = jax.lax.dot_general(
        q_blk, fc1_w_ref[...], (((1,), (1,)), ((), ())),
        preferred_element_type=jnp.float32) + fc1_b_ref[...]


@functools.partial(jax.jit, static_argnums=())
def kernel(z, fc0_w, fc0_b, fc1_w, fc1_b, emb):
    n_batch, n_seq, d_model = z.shape
    n_tok = n_batch * n_seq
    n_rows = n_tok * N_CHANNEL
    z2 = z.reshape(n_tok, d_model)
    grid = (n_tok // _TOK_BLK,)

    bits = jnp.asarray(_threefry_bits_table(n_rows, N_E))

    zq, loss_sum = pl.pallas_call(
        _vq_kernel,
        grid=grid,
        in_specs=[
            pl.BlockSpec((_ROW_BLK, N_E), lambda i: (i, 0)),
            pl.BlockSpec((_TOK_BLK, d_model), lambda i: (i, 0)),
            pl.BlockSpec((D_MODEL, D_MODEL), lambda i: (0, 0)),
            pl.BlockSpec((1, D_MODEL), lambda i: (0, 0)),
            pl.BlockSpec((D_MODEL, D_MODEL), lambda i: (0, 0)),
            pl.BlockSpec((1, D_MODEL), lambda i: (0, 0)),
            pl.BlockSpec((N_E, E_DIM), lambda i: (0, 0)),
            pl.BlockSpec((N_E, E_DIM), lambda i: (0, 0)),
        ],
        out_specs=[
            pl.BlockSpec((_TOK_BLK, d_model), lambda i: (i, 0)),
            pl.BlockSpec((1, 1), lambda i: (0, 0)),
        ],
        out_shape=[
            jax.ShapeDtypeStruct((n_tok, d_model), jnp.float32),
            jax.ShapeDtypeStruct((1, 1), jnp.float32),
        ],
        compiler_params=pltpu.CompilerParams(
            dimension_semantics=("arbitrary",),
        ),
    )(bits, z2, fc0_w, fc0_b.reshape(1, -1), fc1_w, fc1_b.reshape(1, -1),
      emb, jnp.float32(-2.0) * emb)

    mean = loss_sum[0, 0] / jnp.float32(n_tok * N_CHANNEL * E_DIM)
    loss = mean + jnp.float32(BETA) * mean
    return (loss, zq.reshape(n_batch, n_seq, d_model))
